# baseline (speedup 1.0000x reference)
import jax
import jax.numpy as jnp
from jax.experimental import pallas as pl
from jax.experimental.pallas import tpu as pltpu

_BLK = 2500


def _dl_body(m_hbm, w_hbm, x_hbm, o_hbm, wv, xb, mb, ob, w_sem, in_sem,
             out_sem):
    n = x_hbm.shape[0]
    nsteps = n // _BLK

    cw = pltpu.make_async_copy(w_hbm, wv, w_sem)
    cw.start()
    cw.wait()

    def x_copy(s, slot):
        return pltpu.make_async_copy(
            x_hbm.at[pl.ds(s * _BLK, _BLK), :], xb.at[slot], in_sem.at[slot])

    def m_copy(s, slot):
        return pltpu.make_async_copy(
            m_hbm.at[pl.ds(s * _BLK, _BLK), :], mb.at[slot], in_sem.at[slot])

    def o_copy(s, slot):
        return pltpu.make_async_copy(
            ob.at[slot], o_hbm.at[pl.ds(s * _BLK, _BLK), :], out_sem.at[slot])

    x_copy(0, 0).start()
    m_copy(0, 0).start()
    for s in range(nsteps):
        slot = s % 2
        if s + 1 < nsteps:
            x_copy(s + 1, 1 - slot).start()
            m_copy(s + 1, 1 - slot).start()
        x_copy(s, slot).wait()
        m_copy(s, slot).wait()
        if s >= 2:
            o_copy(s - 2, slot).wait()
        x = xb[slot]
        m = mb[slot]
        ob[slot] = jnp.where(m != 0, x * wv[...], x)
        o_copy(s, slot).start()
    if nsteps >= 2:
        o_copy(nsteps - 2, (nsteps - 2) % 2).wait()
    o_copy(nsteps - 1, (nsteps - 1) % 2).wait()


def kernel(x, node_mask, deletion_weight):
    n, d = x.shape
    m = node_mask.astype(jnp.int32).reshape(n, 1)
    w = deletion_weight.reshape(1, d)
    assert n % _BLK == 0
    return pl.pallas_call(
        _dl_body,
        in_specs=[
            pl.BlockSpec(memory_space=pl.MemorySpace.ANY),
            pl.BlockSpec(memory_space=pl.MemorySpace.ANY),
            pl.BlockSpec(memory_space=pl.MemorySpace.ANY),
        ],
        out_specs=pl.BlockSpec(memory_space=pl.MemorySpace.ANY),
        out_shape=jax.ShapeDtypeStruct((n, d), x.dtype),
        scratch_shapes=[
            pltpu.VMEM((1, d), jnp.float32),
            pltpu.VMEM((2, _BLK, d), jnp.float32),
            pltpu.VMEM((2, _BLK, 1), jnp.int32),
            pltpu.VMEM((2, _BLK, d), jnp.float32),
            pltpu.SemaphoreType.DMA,
            pltpu.SemaphoreType.DMA((2,)),
            pltpu.SemaphoreType.DMA((2,)),
        ],
    )(m, w, x)


# speedup vs baseline: 1.0718x; 1.0718x over previous
import jax
import jax.numpy as jnp
from jax.experimental import pallas as pl
from jax.experimental.pallas import tpu as pltpu

_BLK = 1250
_NBUF = 4


def _dl_body(m_hbm, w_hbm, x_hbm, o_hbm, wv, xb, mb, ob, w_sem, in_sem,
             out_sem):
    n = x_hbm.shape[0]
    nsteps = n // _BLK

    cw = pltpu.make_async_copy(w_hbm, wv, w_sem)
    cw.start()
    cw.wait()

    def x_copy(s, slot):
        return pltpu.make_async_copy(
            x_hbm.at[pl.ds(s * _BLK, _BLK), :], xb.at[slot], in_sem.at[slot])

    def m_copy(s, slot):
        return pltpu.make_async_copy(
            m_hbm.at[pl.ds(s * _BLK, _BLK), :], mb.at[slot], in_sem.at[slot])

    def o_copy(s, slot):
        return pltpu.make_async_copy(
            ob.at[slot], o_hbm.at[pl.ds(s * _BLK, _BLK), :], out_sem.at[slot])

    for t in range(min(_NBUF - 1, nsteps)):
        x_copy(t, t).start()
        m_copy(t, t).start()
    for s in range(nsteps):
        slot = s % _NBUF
        pf = s + _NBUF - 1
        if pf < nsteps:
            x_copy(pf, pf % _NBUF).start()
            m_copy(pf, pf % _NBUF).start()
        x_copy(s, slot).wait()
        m_copy(s, slot).wait()
        if s >= _NBUF:
            o_copy(s - _NBUF, slot).wait()
        x = xb[slot]
        m = mb[slot]
        ob[slot] = jnp.where(m != 0, x * wv[...], x)
        o_copy(s, slot).start()
    for t in range(max(0, nsteps - _NBUF), nsteps):
        o_copy(t, t % _NBUF).wait()


def kernel(x, node_mask, deletion_weight):
    n, d = x.shape
    m = node_mask.astype(jnp.int32).reshape(n, 1)
    w = deletion_weight.reshape(1, d)
    assert n % _BLK == 0
    return pl.pallas_call(
        _dl_body,
        in_specs=[
            pl.BlockSpec(memory_space=pl.MemorySpace.ANY),
            pl.BlockSpec(memory_space=pl.MemorySpace.ANY),
            pl.BlockSpec(memory_space=pl.MemorySpace.ANY),
        ],
        out_specs=pl.BlockSpec(memory_space=pl.MemorySpace.ANY),
        out_shape=jax.ShapeDtypeStruct((n, d), x.dtype),
        scratch_shapes=[
            pltpu.VMEM((1, d), jnp.float32),
            pltpu.VMEM((_NBUF, _BLK, d), jnp.float32),
            pltpu.VMEM((_NBUF, _BLK, 1), jnp.int32),
            pltpu.VMEM((_NBUF, _BLK, d), jnp.float32),
            pltpu.SemaphoreType.DMA,
            pltpu.SemaphoreType.DMA((_NBUF,)),
            pltpu.SemaphoreType.DMA((_NBUF,)),
        ],
    )(m, w, x)
